# baseline (speedup 1.0000x reference)
import functools

import jax
import jax.numpy as jnp
from jax import lax
from jax.experimental import pallas as pl
from jax.experimental.pallas import tpu as pltpu
from jax.experimental.pallas import tpu_sc as plsc


@functools.lru_cache(maxsize=None)
def _make_pooling_kernel(B, L, V, D):
    info = plsc.get_sparse_core_info()
    NC, NS, NL = info.num_cores, info.num_subcores, info.num_lanes
    NW = NC * NS
    assert B % NW == 0 and D % (2 * NL) == 0
    b_per_w = B // NW
    NP = D // (2 * NL)
    inv_l = 1.0 / L

    chunks = []
    off = 0
    while off < L:
        sz = min(128, L - off)
        chunks.append((off, sz))
        off += sz

    U = 8
    while L % U:
        U -= 1

    mesh = plsc.VectorSubcoreMesh(core_axis_name="c", subcore_axis_name="s")

    @functools.partial(
        pl.kernel,
        mesh=mesh,
        compiler_params=pltpu.CompilerParams(
            use_tc_tiling_on_sc=False, needs_layout_passes=False
        ),
        out_type=jax.ShapeDtypeStruct((B, D), jnp.float32),
        scratch_types=[
            pltpu.VMEM((b_per_w, L), jnp.int32),
            pltpu.VMEM((2, L, D // 2), jnp.int32),
            pltpu.VMEM((b_per_w, D), jnp.float32),
            pltpu.VMEM((2, NL), jnp.float32),
            pltpu.SemaphoreType.DMA,
            pltpu.SemaphoreType.DMA,
        ],
    )
    def pooled(x_hbm, emb_hbm, out_hbm, idx_v, rows_v, out_v, tmp_v, sem0, sem1):
        wid = lax.axis_index("s") * NC + lax.axis_index("c")
        base = wid * b_per_w
        sems = (sem0, sem1)

        pltpu.sync_copy(x_hbm.at[pl.ds(base, b_per_w)], idx_v)

        def gather_descs(b, slot):
            return [
                pltpu.make_async_copy(
                    emb_hbm.at[idx_v.at[b, pl.ds(off, sz)]],
                    rows_v.at[slot].at[pl.ds(off, sz)],
                    sems[slot],
                )
                for off, sz in chunks
            ]

        def issue(b, slot):
            for cp in gather_descs(b, slot):
                cp.start()

        def drain(b, slot):
            for cp in gather_descs(b, slot):
                cp.wait()

        himask = jnp.full((NL,), jnp.int32(-65536))

        def accum_row(slot, b):
            def j_body(j, accs):
                new = list(accs)
                for u in range(U):
                    jj = j * U + u
                    for p in range(NP):
                        w = rows_v[slot, jj, pl.ds(p * NL, NL)]
                        ev = lax.bitcast_convert_type(
                            lax.shift_left(w, 16), jnp.float32)
                        od = lax.bitcast_convert_type(
                            lax.bitwise_and(w, himask), jnp.float32)
                        new[2 * p] = new[2 * p] + ev
                        new[2 * p + 1] = new[2 * p + 1] + od
                return tuple(new)

            accs = lax.fori_loop(
                0, L // U, j_body,
                tuple(jnp.zeros((NL,), jnp.float32) for _ in range(2 * NP)),
            )

            lane = lax.iota(jnp.int32, NL)
            row_sel = lax.bitwise_and(lane, 1)
            pair_lo = lax.shift_right_logical(lane, 1)
            pair_hi = pair_lo + NL // 2
            for p in range(NP):
                tmp_v[0, :] = accs[2 * p]
                tmp_v[1, :] = accs[2 * p + 1]
                lo = plsc.load_gather(tmp_v, [row_sel, pair_lo])
                hi = plsc.load_gather(tmp_v, [row_sel, pair_hi])
                out_v[b, pl.ds(p * 2 * NL, NL)] = lo * inv_l
                out_v[b, pl.ds(p * 2 * NL + NL, NL)] = hi * inv_l

        issue(0, 0)

        def b_body(i, carry):
            b0 = 2 * i
            issue(b0 + 1, 1)
            drain(b0, 0)
            accum_row(0, b0)

            @pl.when(b0 + 2 < b_per_w)
            def _():
                issue(b0 + 2, 0)

            drain(b0 + 1, 1)
            accum_row(1, b0 + 1)
            return carry

        lax.fori_loop(0, b_per_w // 2, b_body, 0)
        pltpu.sync_copy(out_v, out_hbm.at[pl.ds(base, b_per_w)])

    return pooled


def kernel(x, embedding):
    B, L = x.shape
    V, D = embedding.shape
    pooled = _make_pooling_kernel(B, L, V, D)
    b = embedding.astype(jnp.bfloat16)
    b4 = jnp.concatenate(
        [b[0::4, :], b[1::4, :], b[2::4, :], b[3::4, :]], axis=1
    )
    w = lax.bitcast_convert_type(b4.reshape(V // 4, 2 * D, 2), jnp.int32)
    return pooled(x.astype(jnp.int32), w.reshape(V, D // 2))


# speedup vs baseline: 3.9747x; 3.9747x over previous
import functools

import jax
import jax.numpy as jnp
from jax import lax
from jax.experimental import pallas as pl
from jax.experimental.pallas import tpu as pltpu
from jax.experimental.pallas import tpu_sc as plsc


def _relayout_body(in_ref, out_ref):
    t = in_ref[...].T
    w = t.shape[0]
    t3 = t.reshape(w // 2, 2, t.shape[1])
    out_ref[:, 0:64] = t3[:, 0, :]
    out_ref[:, 64:128] = t3[:, 1, :]


@functools.lru_cache(maxsize=None)
def _make_relayout(V, D, W=512):
    grid = (V + W - 1) // W

    return pl.pallas_call(
        _relayout_body,
        grid=(grid,),
        in_specs=[pl.BlockSpec((D, W), lambda g: (0, g))],
        out_specs=pl.BlockSpec((W // 2, 2 * D), lambda g: (g, 0)),
        out_shape=jax.ShapeDtypeStruct((V // 2, 2 * D), jnp.float32),
    )


@functools.lru_cache(maxsize=None)
def _make_pooling_kernel(B, L, V, D):
    info = plsc.get_sparse_core_info()
    NC, NS, NL = info.num_cores, info.num_subcores, info.num_lanes
    NW = NC * NS
    assert B % NW == 0 and D % NL == 0
    b_per_w = B // NW
    ND = D // NL
    inv_l = 1.0 / L

    chunks = []
    off = 0
    while off < L:
        sz = min(128, L - off)
        chunks.append((off, sz))
        off += sz

    U = 8
    while L % U:
        U -= 1

    mesh = plsc.VectorSubcoreMesh(core_axis_name="c", subcore_axis_name="s")

    @functools.partial(
        pl.kernel,
        mesh=mesh,
        compiler_params=pltpu.CompilerParams(use_tc_tiling_on_sc=False),
        out_type=jax.ShapeDtypeStruct((B, D), jnp.float32),
        scratch_types=[
            pltpu.VMEM((b_per_w, L), jnp.int32),
            pltpu.VMEM((2, L, D), jnp.float32),
            pltpu.VMEM((b_per_w, D), jnp.float32),
            pltpu.SemaphoreType.DMA,
            pltpu.SemaphoreType.DMA,
        ],
    )
    def pooled(x_hbm, emb_hbm, out_hbm, idx_v, rows_v, out_v, sem0, sem1):
        wid = lax.axis_index("s") * NC + lax.axis_index("c")
        base = wid * b_per_w
        sems = (sem0, sem1)

        pltpu.sync_copy(x_hbm.at[pl.ds(base, b_per_w)], idx_v)

        def gather_descs(b, slot):
            return [
                pltpu.make_async_copy(
                    emb_hbm.at[idx_v.at[b, pl.ds(off, sz)]],
                    rows_v.at[slot].at[pl.ds(off, sz)],
                    sems[slot],
                )
                for off, sz in chunks
            ]

        def issue(b, slot):
            for cp in gather_descs(b, slot):
                cp.start()

        def drain(b, slot):
            for cp in gather_descs(b, slot):
                cp.wait()

        def accum_row(slot, b):
            def j_body(j, accs):
                new = list(accs)
                for u in range(U):
                    jj = j * U + u
                    for d in range(ND):
                        new[d] = new[d] + rows_v[slot, jj, pl.ds(d * NL, NL)]
                return tuple(new)

            accs = lax.fori_loop(
                0, L // U, j_body,
                tuple(jnp.zeros((NL,), jnp.float32) for _ in range(ND)),
            )
            for d in range(ND):
                out_v[b, pl.ds(d * NL, NL)] = accs[d] * inv_l

        issue(0, 0)

        def b_body(i, carry):
            b0 = 2 * i
            issue(b0 + 1, 1)
            drain(b0, 0)
            accum_row(0, b0)

            @pl.when(b0 + 2 < b_per_w)
            def _():
                issue(b0 + 2, 0)

            drain(b0 + 1, 1)
            accum_row(1, b0 + 1)
            return carry

        lax.fori_loop(0, b_per_w // 2, b_body, 0)
        pltpu.sync_copy(out_v, out_hbm.at[pl.ds(base, b_per_w)])

    return pooled


def kernel(x, embedding):
    B, L = x.shape
    V, D = embedding.shape
    relayout = _make_relayout(V, D)
    emb_lin = relayout(embedding.T).reshape(V, D)
    pooled = _make_pooling_kernel(B, L, V, D)
    return pooled(x.astype(jnp.int32), emb_lin)


# speedup vs baseline: 7.8969x; 1.9868x over previous
import functools

import jax
import jax.numpy as jnp
from jax import lax
from jax.experimental import pallas as pl
from jax.experimental.pallas import tpu as pltpu
from jax.experimental.pallas import tpu_sc as plsc


def _relayout_body(in_ref, out_ref):
    x = in_ref[...]
    eye = jnp.eye(x.shape[0], dtype=jnp.float32)
    t = lax.dot_general(
        x, eye, (((0,), (0,)), ((), ())),
        preferred_element_type=jnp.float32,
    )
    t3 = t.reshape(t.shape[0] // 2, 2, t.shape[1])
    out_ref[:, 0:64] = t3[:, 0, :]
    out_ref[:, 64:128] = t3[:, 1, :]


@functools.lru_cache(maxsize=None)
def _make_relayout(V, D, W=2048):
    grid = (V + W - 1) // W

    return pl.pallas_call(
        _relayout_body,
        grid=(grid,),
        in_specs=[pl.BlockSpec((D, W), lambda g: (0, g))],
        out_specs=pl.BlockSpec((W // 2, 2 * D), lambda g: (g, 0)),
        out_shape=jax.ShapeDtypeStruct((V // 2, 2 * D), jnp.float32),
    )


@functools.lru_cache(maxsize=None)
def _make_pooling_kernel(B, L, V, D):
    info = plsc.get_sparse_core_info()
    NC, NS, NL = info.num_cores, info.num_subcores, info.num_lanes
    NW = NC * NS
    assert B % NW == 0 and D % NL == 0
    b_per_w = B // NW
    ND = D // NL
    inv_l = 1.0 / L

    chunks = []
    off = 0
    while off < L:
        sz = min(128, L - off)
        chunks.append((off, sz))
        off += sz

    U = 8
    while L % U:
        U -= 1

    mesh = plsc.VectorSubcoreMesh(core_axis_name="c", subcore_axis_name="s")

    @functools.partial(
        pl.kernel,
        mesh=mesh,
        compiler_params=pltpu.CompilerParams(use_tc_tiling_on_sc=False),
        out_type=jax.ShapeDtypeStruct((B, D), jnp.float32),
        scratch_types=[
            pltpu.VMEM((b_per_w, L), jnp.int32),
            pltpu.VMEM((2, L, D), jnp.float32),
            pltpu.VMEM((b_per_w, D), jnp.float32),
            pltpu.SemaphoreType.DMA,
            pltpu.SemaphoreType.DMA,
        ],
    )
    def pooled(x_hbm, emb_hbm, out_hbm, idx_v, rows_v, out_v, sem0, sem1):
        wid = lax.axis_index("s") * NC + lax.axis_index("c")
        base = wid * b_per_w
        sems = (sem0, sem1)

        pltpu.sync_copy(x_hbm.at[pl.ds(base, b_per_w)], idx_v)

        def gather_descs(b, slot):
            return [
                pltpu.make_async_copy(
                    emb_hbm.at[idx_v.at[b, pl.ds(off, sz)]],
                    rows_v.at[slot].at[pl.ds(off, sz)],
                    sems[slot],
                )
                for off, sz in chunks
            ]

        def issue(b, slot):
            for cp in gather_descs(b, slot):
                cp.start()

        def drain(b, slot):
            for cp in gather_descs(b, slot):
                cp.wait()

        def accum_row(slot, b):
            def j_body(j, accs):
                new = list(accs)
                for u in range(U):
                    jj = j * U + u
                    for d in range(ND):
                        new[d] = new[d] + rows_v[slot, jj, pl.ds(d * NL, NL)]
                return tuple(new)

            accs = lax.fori_loop(
                0, L // U, j_body,
                tuple(jnp.zeros((NL,), jnp.float32) for _ in range(ND)),
            )
            for d in range(ND):
                out_v[b, pl.ds(d * NL, NL)] = accs[d] * inv_l

        issue(0, 0)

        def b_body(i, carry):
            b0 = 2 * i
            issue(b0 + 1, 1)
            drain(b0, 0)
            accum_row(0, b0)

            @pl.when(b0 + 2 < b_per_w)
            def _():
                issue(b0 + 2, 0)

            drain(b0 + 1, 1)
            accum_row(1, b0 + 1)
            return carry

        lax.fori_loop(0, b_per_w // 2, b_body, 0)
        pltpu.sync_copy(out_v, out_hbm.at[pl.ds(base, b_per_w)])

    return pooled


def kernel(x, embedding):
    B, L = x.shape
    V, D = embedding.shape
    relayout = _make_relayout(V, D)
    emb_lin = relayout(embedding.T).reshape(V, D)
    pooled = _make_pooling_kernel(B, L, V, D)
    return pooled(x.astype(jnp.int32), emb_lin)


# speedup vs baseline: 9.7019x; 1.2286x over previous
import functools

import jax
import jax.numpy as jnp
from jax import lax
from jax.experimental import pallas as pl
from jax.experimental.pallas import tpu as pltpu
from jax.experimental.pallas import tpu_sc as plsc


def _relayout_body(in_ref, out_ref):
    x = in_ref[...].astype(jnp.bfloat16)
    eye = jnp.eye(x.shape[0], dtype=jnp.bfloat16)
    t = lax.dot_general(
        x, eye, (((0,), (0,)), ((), ())),
        preferred_element_type=jnp.float32,
    )
    t3 = t.reshape(t.shape[0] // 2, 2, t.shape[1])
    out_ref[:, 0:64] = t3[:, 0, :]
    out_ref[:, 64:128] = t3[:, 1, :]


@functools.lru_cache(maxsize=None)
def _make_relayout(V, D, W=4096):
    grid = (V + W - 1) // W

    return pl.pallas_call(
        _relayout_body,
        grid=(grid,),
        in_specs=[pl.BlockSpec((D, W), lambda g: (0, g))],
        out_specs=pl.BlockSpec((W // 2, 2 * D), lambda g: (g, 0)),
        out_shape=jax.ShapeDtypeStruct((V // 2, 2 * D), jnp.float32),
    )


@functools.lru_cache(maxsize=None)
def _make_pooling_kernel(B, L, V, D):
    info = plsc.get_sparse_core_info()
    NC, NS, NL = info.num_cores, info.num_subcores, info.num_lanes
    NW = NC * NS
    assert B % NW == 0 and D % NL == 0
    b_per_w = B // NW
    ND = D // NL
    inv_l = 1.0 / L

    chunks = []
    off = 0
    while off < L:
        sz = min(128, L - off)
        chunks.append((off, sz))
        off += sz

    U = 8
    while L % U:
        U -= 1

    mesh = plsc.VectorSubcoreMesh(core_axis_name="c", subcore_axis_name="s")

    @functools.partial(
        pl.kernel,
        mesh=mesh,
        compiler_params=pltpu.CompilerParams(use_tc_tiling_on_sc=False),
        out_type=jax.ShapeDtypeStruct((B, D), jnp.float32),
        scratch_types=[
            pltpu.VMEM((b_per_w, L), jnp.int32),
            pltpu.VMEM((2, L, D), jnp.float32),
            pltpu.VMEM((b_per_w, D), jnp.float32),
            pltpu.SemaphoreType.DMA,
            pltpu.SemaphoreType.DMA,
        ],
    )
    def pooled(x_hbm, emb_hbm, out_hbm, idx_v, rows_v, out_v, sem0, sem1):
        wid = lax.axis_index("s") * NC + lax.axis_index("c")
        base = wid * b_per_w
        sems = (sem0, sem1)

        pltpu.sync_copy(x_hbm.at[pl.ds(base, b_per_w)], idx_v)

        def gather_descs(b, slot):
            return [
                pltpu.make_async_copy(
                    emb_hbm.at[idx_v.at[b, pl.ds(off, sz)]],
                    rows_v.at[slot].at[pl.ds(off, sz)],
                    sems[slot],
                )
                for off, sz in chunks
            ]

        def issue(b, slot):
            for cp in gather_descs(b, slot):
                cp.start()

        def drain(b, slot):
            for cp in gather_descs(b, slot):
                cp.wait()

        def accum_row(slot, b):
            def j_body(j, accs):
                new = list(accs)
                for u in range(U):
                    jj = j * U + u
                    for d in range(ND):
                        new[d] = new[d] + rows_v[slot, jj, pl.ds(d * NL, NL)]
                return tuple(new)

            accs = lax.fori_loop(
                0, L // U, j_body,
                tuple(jnp.zeros((NL,), jnp.float32) for _ in range(ND)),
            )
            for d in range(ND):
                out_v[b, pl.ds(d * NL, NL)] = accs[d] * inv_l

        issue(0, 0)

        def b_body(i, carry):
            b0 = 2 * i
            issue(b0 + 1, 1)
            drain(b0, 0)
            accum_row(0, b0)

            @pl.when(b0 + 2 < b_per_w)
            def _():
                issue(b0 + 2, 0)

            drain(b0 + 1, 1)
            accum_row(1, b0 + 1)
            return carry

        lax.fori_loop(0, b_per_w // 2, b_body, 0)
        pltpu.sync_copy(out_v, out_hbm.at[pl.ds(base, b_per_w)])

    return pooled


def kernel(x, embedding):
    B, L = x.shape
    V, D = embedding.shape
    relayout = _make_relayout(V, D)
    emb_lin = relayout(embedding.T).reshape(V, D)
    pooled = _make_pooling_kernel(B, L, V, D)
    return pooled(x.astype(jnp.int32), emb_lin)
